# Initial kernel scaffold; baseline (speedup 1.0000x reference)
import jax
import jax.numpy as jnp
from jax.experimental import pallas as pl


def kernel(nxyz, nbr_list, weights, atom_embed, ef1_W, ef1_b, ef2_W, ef2_b, nf_W, nf_b, up1_W, up1_b, up2_W, up2_b, mol_W1, mol_b1, mol_W2, mol_b2, ro_W1, ro_b1, ro_W2, ro_b2):
    raise NotImplementedError("write your pallas kernel here")


# speedup vs baseline: 2.4240x; 2.4240x over previous
import functools

import jax
import jax.numpy as jnp
from jax import lax
from jax.experimental import pallas as pl
from jax.experimental.pallas import tpu as pltpu
from jax.experimental.pallas import tpu_sc as plsc

N_ATOMS = 10000
N_EDGES = 160000
N_SPECIES = 50
N_CONFS = 10
MOL_SIZE = 20
F = 128
G = 32
N_CONV = 3
CUTOFF = 5.0

NC = 2
NS = 16
NW = NC * NS
EW = N_EDGES // NW
EWP = 5008
EP = 163840
EWM = EP // NW
C = 128
NCH = EWM // C
NP = 10112
RPS = NP // NS

_LOG2 = 0.6931471805599453


def _ssp(x):
    return jnp.maximum(x, 0.0) + jnp.log(1.0 + jnp.exp(-jnp.abs(x))) - _LOG2


def _mesh():
    return plsc.VectorSubcoreMesh(
        core_axis_name="c", subcore_axis_name="s", num_cores=NC, num_subcores=NS
    )


def _sc_d2_body(x_hbm, y_hbm, z_hbm, a0_hbm, a1_hbm, out_hbm,
                xv, yv, zv, i0v, i1v, dv):
    cid = lax.axis_index("c")
    sid = lax.axis_index("s")
    wid = cid * NS + sid
    pltpu.sync_copy(x_hbm, xv)
    pltpu.sync_copy(y_hbm, yv)
    pltpu.sync_copy(z_hbm, zv)
    base = wid * EW
    pltpu.sync_copy(a0_hbm.at[pl.ds(base, EWP)], i0v)
    pltpu.sync_copy(a1_hbm.at[pl.ds(base, EWP)], i1v)

    def body(k, _):
        i0 = i0v[pl.ds(k * 16, 16)]
        i1 = i1v[pl.ds(k * 16, 16)]
        dx = plsc.load_gather(xv, [i0]) - plsc.load_gather(xv, [i1])
        dy = plsc.load_gather(yv, [i0]) - plsc.load_gather(yv, [i1])
        dz = plsc.load_gather(zv, [i0]) - plsc.load_gather(zv, [i1])
        dv[pl.ds(k * 16, 16)] = dx * dx + dy * dy + dz * dz
        return 0

    lax.fori_loop(0, EWP // 16, body, 0)
    pltpu.sync_copy(dv, out_hbm.at[wid])


def _sc_d2(x, y, z, a0p, a1p):
    fn = pl.kernel(
        _sc_d2_body,
        out_type=jax.ShapeDtypeStruct((NW, EWP), jnp.float32),
        mesh=_mesh(),
        scratch_types=[
            pltpu.VMEM((N_ATOMS,), jnp.float32),
            pltpu.VMEM((N_ATOMS,), jnp.float32),
            pltpu.VMEM((N_ATOMS,), jnp.float32),
            pltpu.VMEM((EWP,), jnp.int32),
            pltpu.VMEM((EWP,), jnp.int32),
            pltpu.VMEM((EWP,), jnp.float32),
        ],
        compiler_params=pltpu.CompilerParams(needs_layout_passes=False),
    )
    return fn(x, y, z, a0p, a1p)


def _sc_msg_body(nodes_hbm, we_hbm, idx0_hbm, idx1_hbm, zeros_hbm, out_hbm,
                 ichunk0, ichunk1, rows0, rows1, web, agg, sem0, sem1, sem2):
    cid = lax.axis_index("c")
    sid = lax.axis_index("s")
    wid = cid * NS + sid

    pltpu.sync_copy(zeros_hbm, agg.at[pl.ds(sid * RPS, RPS)])
    plsc.subcore_barrier()

    def chunk(j, _):
        pltpu.sync_copy(idx0_hbm.at[wid, j], ichunk0)
        pltpu.sync_copy(idx1_hbm.at[wid, j], ichunk1)
        d0 = pltpu.async_copy(nodes_hbm.at[ichunk0], rows0, sem0)
        d1 = pltpu.async_copy(nodes_hbm.at[ichunk1], rows1, sem1)
        d2 = pltpu.async_copy(we_hbm.at[pl.ds(wid * EWM + j * C, C)], web, sem2)
        d0.wait()
        d1.wait()
        d2.wait()

        def mrow(r, _):
            for v in range(8):
                w16 = web[r, pl.ds(v * 16, 16)]
                rows0[r, pl.ds(v * 16, 16)] = rows0[r, pl.ds(v * 16, 16)] * w16
                rows1[r, pl.ds(v * 16, 16)] = rows1[r, pl.ds(v * 16, 16)] * w16
            return 0

        lax.fori_loop(0, C, mrow, 0)
        pltpu.sync_copy(rows0, agg.at[ichunk1], add=True)
        pltpu.sync_copy(rows1, agg.at[ichunk0], add=True)
        return 0

    lax.fori_loop(0, NCH, chunk, 0)
    plsc.subcore_barrier()
    pltpu.sync_copy(agg.at[pl.ds(sid * RPS, RPS)],
                    out_hbm.at[cid, pl.ds(sid * RPS, RPS)])


def _sc_msg(nodes, we, idx0_slab, idx1_slab, zeros):
    fn = pl.kernel(
        _sc_msg_body,
        out_type=jax.ShapeDtypeStruct((NC, NP, F), jnp.float32),
        mesh=_mesh(),
        scratch_types=[
            pltpu.VMEM((C,), jnp.int32),
            pltpu.VMEM((C,), jnp.int32),
            pltpu.VMEM((C, F), jnp.float32),
            pltpu.VMEM((C, F), jnp.float32),
            pltpu.VMEM((C, F), jnp.float32),
            pltpu.VMEM_SHARED((NP, F), jnp.float32),
            pltpu.SemaphoreType.DMA,
            pltpu.SemaphoreType.DMA,
            pltpu.SemaphoreType.DMA,
        ],
        compiler_params=pltpu.CompilerParams(needs_layout_passes=False),
    )
    return fn(nodes, we, idx0_slab, idx1_slab, zeros)


_EB = 1024
_AB = 1000


def _tc_we_body(d2_ref, ef1w_ref, ef1b_ref, ef2w_ref, ef2b_ref, o0, o1, o2):
    b = pl.program_id(0)
    d = jnp.sqrt(d2_ref[...])
    width = CUTOFF / (G - 1)
    coeff = -0.5 / (width * width)
    offs = lax.broadcasted_iota(jnp.int32, (_EB, G), 1).astype(jnp.float32) * width
    g = jnp.exp(coeff * (d - offs) ** 2)
    rows = b * _EB + lax.broadcasted_iota(jnp.int32, (_EB, 1), 0)
    valid = (rows < N_EDGES).astype(jnp.float32)
    outs = (o0, o1, o2)
    for i in range(N_CONV):
        h = _ssp(jnp.dot(g, ef1w_ref[i], preferred_element_type=jnp.float32)
                 + ef1b_ref[i])
        outs[i][...] = (jnp.dot(h, ef2w_ref[i], preferred_element_type=jnp.float32)
                        + ef2b_ref[i]) * valid


def _tc_we(d2, ef1_W, ef1_b, ef2_W, ef2_b):
    nblk = EP // _EB
    full = lambda s: pl.BlockSpec(s, lambda i: (0,) * len(s))
    out = jax.ShapeDtypeStruct((EP, F), jnp.float32)
    return pl.pallas_call(
        _tc_we_body,
        grid=(nblk,),
        in_specs=[
            pl.BlockSpec((_EB, 1), lambda i: (i, 0)),
            full((N_CONV, G, F)), full((N_CONV, 1, F)),
            full((N_CONV, F, F)), full((N_CONV, 1, F)),
        ],
        out_specs=[pl.BlockSpec((_EB, F), lambda i: (i, 0))] * 3,
        out_shape=[out, out, out],
    )(d2, ef1_W, ef1_b, ef2_W, ef2_b)


def _tc_embed_body(z_ref, emb_ref, nfw_ref, nfb_ref, r_out, nodes_out):
    zb = z_ref[...]
    onehot = (zb == lax.broadcasted_iota(jnp.int32, (_AB, F), 1)).astype(jnp.float32)
    r = jnp.dot(onehot, emb_ref[...], preferred_element_type=jnp.float32)
    r_out[...] = r
    nodes_out[...] = (jnp.dot(r, nfw_ref[0], preferred_element_type=jnp.float32)
                      + nfb_ref[0])


def _tc_embed(z, emb_pad, nf_W, nf_b):
    nblk = N_ATOMS // _AB
    full = lambda s: pl.BlockSpec(s, lambda i: (0,) * len(s))
    out = jax.ShapeDtypeStruct((N_ATOMS, F), jnp.float32)
    return pl.pallas_call(
        _tc_embed_body,
        grid=(nblk,),
        in_specs=[
            pl.BlockSpec((_AB, 1), lambda i: (i, 0)),
            full((F, F)), full((N_CONV, F, F)), full((N_CONV, 1, F)),
        ],
        out_specs=[pl.BlockSpec((_AB, F), lambda i: (i, 0))] * 2,
        out_shape=[out, out],
    )(z, emb_pad, nf_W, nf_b)


def _tc_update_body(i, want_nodes, r_ref, q0_ref, q1_ref,
                    up1w_ref, up1b_ref, up2w_ref, up2b_ref,
                    nfw_ref, nfb_ref, *outs):
    agg = q0_ref[...] + q1_ref[...]
    h = _ssp(jnp.dot(agg, up1w_ref[i], preferred_element_type=jnp.float32)
             + up1b_ref[i])
    dr = jnp.dot(h, up2w_ref[i], preferred_element_type=jnp.float32) + up2b_ref[i]
    rn = r_ref[...] + dr
    outs[0][...] = rn
    if want_nodes:
        outs[1][...] = (jnp.dot(rn, nfw_ref[i + 1], preferred_element_type=jnp.float32)
                        + nfb_ref[i + 1])


def _tc_update(i, want_nodes, r, q0, q1, up1_W, up1_b, up2_W, up2_b, nf_W, nf_b):
    nblk = N_ATOMS // _AB
    full = lambda s: pl.BlockSpec(s, lambda i: (0,) * len(s))
    ab = pl.BlockSpec((_AB, F), lambda i: (i, 0))
    out = jax.ShapeDtypeStruct((N_ATOMS, F), jnp.float32)
    return pl.pallas_call(
        functools.partial(_tc_update_body, i, want_nodes),
        grid=(nblk,),
        in_specs=[ab, ab, ab,
                  full((N_CONV, F, F)), full((N_CONV, 1, F)),
                  full((N_CONV, F, F)), full((N_CONV, 1, F)),
                  full((N_CONV, F, F)), full((N_CONV, 1, F))],
        out_specs=[ab] * (2 if want_nodes else 1),
        out_shape=[out] * (2 if want_nodes else 1),
    )(r, q0, q1, up1_W, up1_b, up2_W, up2_b, nf_W, nf_b)


def _tc_readout_body(r2_ref, S_ref, mw1_ref, mb1_ref, mw2_ref, mb2_ref,
                     wcol_ref, P_ref, rw1_ref, rb1_ref, rw2_ref, rb2_ref, o_ref):
    pooled = jnp.dot(r2_ref[...], S_ref[...], preferred_element_type=jnp.float32)
    h = _ssp(jnp.dot(pooled, mw1_ref[...], preferred_element_type=jnp.float32)
             + mb1_ref[...])
    mol = jnp.dot(h, mw2_ref[...], preferred_element_type=jnp.float32) + mb2_ref[...]
    molw = mol * wcol_ref[...]
    fp = jnp.dot(P_ref[...], molw, preferred_element_type=jnp.float32)
    hh = _ssp(jnp.dot(fp, rw1_ref[...], preferred_element_type=jnp.float32)
              + rb1_ref[...])
    o_ref[...] = jnp.dot(hh, rw2_ref[...], preferred_element_type=jnp.float32) + rb2_ref[...]


def _tc_readout(r2, S, mw1, mb1, mw2, mb2, wcol, P, rw1, rb1, rw2, rb2):
    return pl.pallas_call(
        _tc_readout_body,
        out_shape=jax.ShapeDtypeStruct((N_SPECIES, 1), jnp.float32),
    )(r2, S, mw1, mb1, mw2, mb2, wcol, P, rw1, rb1, rw2, rb2)


def kernel(nxyz, nbr_list, weights, atom_embed,
           ef1_W, ef1_b, ef2_W, ef2_b, nf_W, nf_b,
           up1_W, up1_b, up2_W, up2_b,
           mol_W1, mol_b1, mol_W2, mol_b2,
           ro_W1, ro_b1, ro_W2, ro_b2):
    f32 = jnp.float32
    z = nxyz[:, 0].astype(jnp.int32).reshape(N_ATOMS, 1)
    xyz = nxyz[:, 1:4]
    a0 = nbr_list[:, 0].astype(jnp.int32)
    a1 = nbr_list[:, 1].astype(jnp.int32)
    a0p = jnp.pad(a0, (0, EWP - EW))
    a1p = jnp.pad(a1, (0, EWP - EW))
    idx0_slab = jnp.pad(a0, (0, EP - N_EDGES)).reshape(NW, NCH, C)
    idx1_slab = jnp.pad(a1, (0, EP - N_EDGES)).reshape(NW, NCH, C)
    zeros_blk = jnp.zeros((RPS, F), f32)

    emb_pad = jnp.zeros((F, F), f32).at[:100].set(atom_embed)
    b3 = lambda b: b.reshape(N_CONV, 1, F)
    ef1b, ef2b, nfb = b3(ef1_b), b3(ef2_b), b3(nf_b)
    up1b, up2b = b3(up1_b), b3(up2_b)

    d2w = _sc_d2(xyz[:, 0].ravel(), xyz[:, 1].ravel(), xyz[:, 2].ravel(), a0p, a1p)
    d2 = jnp.pad(d2w[:, :EW].reshape(N_EDGES), (0, EP - N_EDGES)).reshape(EP, 1)

    we0, we1, we2 = _tc_we(d2, ef1_W, ef1b, ef2_W, ef2b)
    wes = (we0, we1, we2)

    r, nodes = _tc_embed(z, emb_pad, nf_W, nfb)

    for i in range(N_CONV):
        part = _sc_msg(nodes, wes[i], idx0_slab, idx1_slab, zeros_blk)
        want_nodes = i < N_CONV - 1
        res = _tc_update(i, want_nodes, r, part[0, :N_ATOMS], part[1, :N_ATOMS],
                         up1_W, up1b, up2_W, up2b, nf_W, nfb)
        if want_nodes:
            r, nodes = res
        else:
            (r,) = res

    r2 = r.reshape(N_SPECIES * N_CONFS, MOL_SIZE * F)
    S = jnp.tile(jnp.eye(F, dtype=f32), (MOL_SIZE, 1))
    P = jnp.kron(jnp.eye(N_SPECIES, dtype=f32), jnp.ones((1, N_CONFS), f32))
    wcol = weights.reshape(N_SPECIES * N_CONFS, 1)
    mb1 = mol_b1.reshape(1, -1)
    mb2 = mol_b2.reshape(1, -1)
    rb1 = ro_b1.reshape(1, -1)
    rb2 = ro_b2.reshape(1, 1)
    return _tc_readout(r2, S, mol_W1, mb1, mol_W2, mb2, wcol, P,
                       ro_W1, rb1, ro_W2, rb2)


# speedup vs baseline: 2.8599x; 1.1798x over previous
import functools

import jax
import jax.numpy as jnp
from jax import lax
from jax.experimental import pallas as pl
from jax.experimental.pallas import tpu as pltpu
from jax.experimental.pallas import tpu_sc as plsc

N_ATOMS = 10000
N_EDGES = 160000
N_SPECIES = 50
N_CONFS = 10
MOL_SIZE = 20
F = 128
G = 32
N_CONV = 3
CUTOFF = 5.0

NC = 2
NS = 16
NW = NC * NS
EP = 163840
EWM = EP // NW
C = 64
TOT_CH = EP // C
N0 = 40
N1 = (TOT_CH - NS * N0) // NS
NCHMAX = max(N0, N1)
NP = 10112
RPS = NP // NS

_LOG2 = 0.6931471805599453


def _ssp(x):
    return jnp.maximum(x, 0.0) + jnp.log(1.0 + jnp.exp(-jnp.abs(x))) - _LOG2


def _mesh():
    return plsc.VectorSubcoreMesh(
        core_axis_name="c", subcore_axis_name="s", num_cores=NC, num_subcores=NS
    )


def _sc_d2_body(x_hbm, y_hbm, z_hbm, a0_hbm, a1_hbm, out_hbm,
                xv, yv, zv, i0v, i1v, dv):
    cid = lax.axis_index("c")
    sid = lax.axis_index("s")
    wid = cid * NS + sid
    pltpu.sync_copy(x_hbm, xv)
    pltpu.sync_copy(y_hbm, yv)
    pltpu.sync_copy(z_hbm, zv)
    base = wid * EWM
    pltpu.sync_copy(a0_hbm.at[pl.ds(base, EWM)], i0v)
    pltpu.sync_copy(a1_hbm.at[pl.ds(base, EWM)], i1v)

    def body(k, _):
        i0 = i0v[pl.ds(k * 16, 16)]
        i1 = i1v[pl.ds(k * 16, 16)]
        dx = plsc.load_gather(xv, [i0]) - plsc.load_gather(xv, [i1])
        dy = plsc.load_gather(yv, [i0]) - plsc.load_gather(yv, [i1])
        dz = plsc.load_gather(zv, [i0]) - plsc.load_gather(zv, [i1])
        dv[pl.ds(k * 16, 16)] = dx * dx + dy * dy + dz * dz
        return 0

    lax.fori_loop(0, EWM // 16, body, 0)
    pltpu.sync_copy(dv, out_hbm.at[wid])


def _sc_d2(x, y, z, a0p, a1p):
    fn = pl.kernel(
        _sc_d2_body,
        out_type=jax.ShapeDtypeStruct((NW, EWM), jnp.float32),
        mesh=_mesh(),
        scratch_types=[
            pltpu.VMEM((N_ATOMS,), jnp.float32),
            pltpu.VMEM((N_ATOMS,), jnp.float32),
            pltpu.VMEM((N_ATOMS,), jnp.float32),
            pltpu.VMEM((EWM,), jnp.int32),
            pltpu.VMEM((EWM,), jnp.int32),
            pltpu.VMEM((EWM,), jnp.float32),
        ],
        compiler_params=pltpu.CompilerParams(needs_layout_passes=False),
    )
    return fn(x, y, z, a0p, a1p)


def _sc_msg_body(nodes_hbm, we_hbm, idx0_hbm, idx1_hbm, zeros_hbm, out_hbm,
                 ic0_0, ic0_1, ic1_0, ic1_1,
                 rows0_0, rows0_1, rows1_0, rows1_1, web_0, web_1, agg,
                 si_0, si_1, sg0_0, sg0_1, sg1_0, sg1_1, sw_0, sw_1):
    cid = lax.axis_index("c")
    sid = lax.axis_index("s")

    ncc = jnp.where(cid == 0, N0, N1)
    cb = jnp.where(cid == 0, sid * N0, NS * N0 + sid * N1)

    ic0 = (ic0_0, ic0_1)
    ic1 = (ic1_0, ic1_1)
    rows0 = (rows0_0, rows0_1)
    rows1 = (rows1_0, rows1_1)
    web = (web_0, web_1)
    si = (si_0, si_1)
    sg0 = (sg0_0, sg0_1)
    sg1 = (sg1_0, sg1_1)
    sw = (sw_0, sw_1)

    def fire_idx(j, b):
        pltpu.async_copy(idx0_hbm.at[pl.ds((cb + j) * C, C)], ic0[b], si[b])
        pltpu.async_copy(idx1_hbm.at[pl.ds((cb + j) * C, C)], ic1[b], si[b])

    def wait_idx(b):
        pltpu.make_async_copy(idx0_hbm.at[pl.ds(0, C)], ic0[b], si[b]).wait()
        pltpu.make_async_copy(idx1_hbm.at[pl.ds(0, C)], ic1[b], si[b]).wait()

    def fire_gathers(j, b):
        pltpu.async_copy(nodes_hbm.at[ic0[b]], rows0[b], sg0[b])
        pltpu.async_copy(nodes_hbm.at[ic1[b]], rows1[b], sg1[b])
        pltpu.async_copy(we_hbm.at[pl.ds((cb + j) * C, C)], web[b], sw[b])

    def wait_gathers(b):
        pltpu.make_async_copy(nodes_hbm.at[ic0[b]], rows0[b], sg0[b]).wait()
        pltpu.make_async_copy(nodes_hbm.at[ic1[b]], rows1[b], sg1[b]).wait()
        pltpu.make_async_copy(we_hbm.at[pl.ds(0, C)], web[b], sw[b]).wait()

    fire_idx(0, 0)
    fire_idx(1, 1)
    wait_idx(0)
    fire_gathers(0, 0)

    pltpu.sync_copy(zeros_hbm, agg.at[pl.ds(sid * RPS, RPS)])
    plsc.subcore_barrier()

    def super_step(t, _):
        for b in range(2):
            j = 2 * t + b
            nb = 1 - b
            wait_gathers(b)

            @pl.when(j + 1 < ncc)
            def _():
                wait_idx(nb)
                fire_gathers(j + 1, nb)

            def mrow(r, _):
                for v in range(8):
                    w16 = web[b][r, pl.ds(v * 16, 16)]
                    rows0[b][r, pl.ds(v * 16, 16)] = rows0[b][r, pl.ds(v * 16, 16)] * w16
                    rows1[b][r, pl.ds(v * 16, 16)] = rows1[b][r, pl.ds(v * 16, 16)] * w16
                return 0

            lax.fori_loop(0, C, mrow, 0)
            pltpu.sync_copy(rows0[b], agg.at[ic1[b]], add=True)
            pltpu.sync_copy(rows1[b], agg.at[ic0[b]], add=True)

            @pl.when(j + 2 < ncc)
            def _():
                fire_idx(j + 2, b)
        return 0

    lax.fori_loop(0, ncc // 2, super_step, 0)
    plsc.subcore_barrier()
    pltpu.sync_copy(agg.at[pl.ds(sid * RPS, RPS)],
                    out_hbm.at[cid, pl.ds(sid * RPS, RPS)])


def _sc_msg(nodes, we, idx0, idx1, zeros):
    fn = pl.kernel(
        _sc_msg_body,
        out_type=jax.ShapeDtypeStruct((NC, NP, F), jnp.float32),
        mesh=_mesh(),
        scratch_types=[
            pltpu.VMEM((C,), jnp.int32),
            pltpu.VMEM((C,), jnp.int32),
            pltpu.VMEM((C,), jnp.int32),
            pltpu.VMEM((C,), jnp.int32),
            pltpu.VMEM((C, F), jnp.float32),
            pltpu.VMEM((C, F), jnp.float32),
            pltpu.VMEM((C, F), jnp.float32),
            pltpu.VMEM((C, F), jnp.float32),
            pltpu.VMEM((C, F), jnp.float32),
            pltpu.VMEM((C, F), jnp.float32),
            pltpu.VMEM_SHARED((NP, F), jnp.float32),
            pltpu.SemaphoreType.DMA,
            pltpu.SemaphoreType.DMA,
            pltpu.SemaphoreType.DMA,
            pltpu.SemaphoreType.DMA,
            pltpu.SemaphoreType.DMA,
            pltpu.SemaphoreType.DMA,
            pltpu.SemaphoreType.DMA,
            pltpu.SemaphoreType.DMA,
        ],
        compiler_params=pltpu.CompilerParams(needs_layout_passes=False),
    )
    return fn(nodes, we, idx0, idx1, zeros)


_EB = 1024
_AB = 1000


def _tc_we_body(d2_ref, valid_ref, ef1w_ref, ef1b_ref, ef2w_ref, ef2b_ref,
                o0, o1, o2):
    d = jnp.sqrt(d2_ref[...])
    width = CUTOFF / (G - 1)
    coeff = -0.5 / (width * width)
    offs = lax.broadcasted_iota(jnp.int32, (_EB, G), 1).astype(jnp.float32) * width
    g = jnp.exp(coeff * (d - offs) ** 2)
    valid = valid_ref[...]
    outs = (o0, o1, o2)
    for i in range(N_CONV):
        h = _ssp(jnp.dot(g, ef1w_ref[i], preferred_element_type=jnp.float32)
                 + ef1b_ref[i])
        outs[i][...] = (jnp.dot(h, ef2w_ref[i], preferred_element_type=jnp.float32)
                        + ef2b_ref[i]) * valid


def _tc_we(d2, valid, ef1_W, ef1_b, ef2_W, ef2_b):
    nblk = EP // _EB
    full = lambda s: pl.BlockSpec(s, lambda i: (0,) * len(s))
    out = jax.ShapeDtypeStruct((EP, F), jnp.float32)
    return pl.pallas_call(
        _tc_we_body,
        grid=(nblk,),
        in_specs=[
            pl.BlockSpec((_EB, 1), lambda i: (i, 0)),
            pl.BlockSpec((_EB, 1), lambda i: (i, 0)),
            full((N_CONV, G, F)), full((N_CONV, 1, F)),
            full((N_CONV, F, F)), full((N_CONV, 1, F)),
        ],
        out_specs=[pl.BlockSpec((_EB, F), lambda i: (i, 0))] * 3,
        out_shape=[out, out, out],
    )(d2, valid, ef1_W, ef1_b, ef2_W, ef2_b)


def _tc_embed_body(z_ref, emb_ref, nfw_ref, nfb_ref, r_out, nodes_out):
    zb = z_ref[...]
    onehot = (zb == lax.broadcasted_iota(jnp.int32, (_AB, F), 1)).astype(jnp.float32)
    r = jnp.dot(onehot, emb_ref[...], preferred_element_type=jnp.float32)
    r_out[...] = r
    nodes_out[...] = (jnp.dot(r, nfw_ref[0], preferred_element_type=jnp.float32)
                      + nfb_ref[0])


def _tc_embed(z, emb_pad, nf_W, nf_b):
    nblk = N_ATOMS // _AB
    full = lambda s: pl.BlockSpec(s, lambda i: (0,) * len(s))
    out = jax.ShapeDtypeStruct((N_ATOMS, F), jnp.float32)
    return pl.pallas_call(
        _tc_embed_body,
        grid=(nblk,),
        in_specs=[
            pl.BlockSpec((_AB, 1), lambda i: (i, 0)),
            full((F, F)), full((N_CONV, F, F)), full((N_CONV, 1, F)),
        ],
        out_specs=[pl.BlockSpec((_AB, F), lambda i: (i, 0))] * 2,
        out_shape=[out, out],
    )(z, emb_pad, nf_W, nf_b)


def _tc_update_body(i, want_nodes, r_ref, q0_ref, q1_ref,
                    up1w_ref, up1b_ref, up2w_ref, up2b_ref,
                    nfw_ref, nfb_ref, *outs):
    agg = q0_ref[...] + q1_ref[...]
    h = _ssp(jnp.dot(agg, up1w_ref[i], preferred_element_type=jnp.float32)
             + up1b_ref[i])
    dr = jnp.dot(h, up2w_ref[i], preferred_element_type=jnp.float32) + up2b_ref[i]
    rn = r_ref[...] + dr
    outs[0][...] = rn
    if want_nodes:
        outs[1][...] = (jnp.dot(rn, nfw_ref[i + 1], preferred_element_type=jnp.float32)
                        + nfb_ref[i + 1])


def _tc_update(i, want_nodes, r, q0, q1, up1_W, up1_b, up2_W, up2_b, nf_W, nf_b):
    nblk = N_ATOMS // _AB
    full = lambda s: pl.BlockSpec(s, lambda i: (0,) * len(s))
    ab = pl.BlockSpec((_AB, F), lambda i: (i, 0))
    out = jax.ShapeDtypeStruct((N_ATOMS, F), jnp.float32)
    return pl.pallas_call(
        functools.partial(_tc_update_body, i, want_nodes),
        grid=(nblk,),
        in_specs=[ab, ab, ab,
                  full((N_CONV, F, F)), full((N_CONV, 1, F)),
                  full((N_CONV, F, F)), full((N_CONV, 1, F)),
                  full((N_CONV, F, F)), full((N_CONV, 1, F))],
        out_specs=[ab] * (2 if want_nodes else 1),
        out_shape=[out] * (2 if want_nodes else 1),
    )(r, q0, q1, up1_W, up1_b, up2_W, up2_b, nf_W, nf_b)


def _tc_readout_body(r2_ref, S_ref, mw1_ref, mb1_ref, mw2_ref, mb2_ref,
                     wcol_ref, P_ref, rw1_ref, rb1_ref, rw2_ref, rb2_ref, o_ref):
    pooled = jnp.dot(r2_ref[...], S_ref[...], preferred_element_type=jnp.float32)
    h = _ssp(jnp.dot(pooled, mw1_ref[...], preferred_element_type=jnp.float32)
             + mb1_ref[...])
    mol = jnp.dot(h, mw2_ref[...], preferred_element_type=jnp.float32) + mb2_ref[...]
    molw = mol * wcol_ref[...]
    fp = jnp.dot(P_ref[...], molw, preferred_element_type=jnp.float32)
    hh = _ssp(jnp.dot(fp, rw1_ref[...], preferred_element_type=jnp.float32)
              + rb1_ref[...])
    o_ref[...] = jnp.dot(hh, rw2_ref[...], preferred_element_type=jnp.float32) + rb2_ref[...]


def _tc_readout(r2, S, mw1, mb1, mw2, mb2, wcol, P, rw1, rb1, rw2, rb2):
    return pl.pallas_call(
        _tc_readout_body,
        out_shape=jax.ShapeDtypeStruct((N_SPECIES, 1), jnp.float32),
    )(r2, S, mw1, mb1, mw2, mb2, wcol, P, rw1, rb1, rw2, rb2)


def kernel(nxyz, nbr_list, weights, atom_embed,
           ef1_W, ef1_b, ef2_W, ef2_b, nf_W, nf_b,
           up1_W, up1_b, up2_W, up2_b,
           mol_W1, mol_b1, mol_W2, mol_b2,
           ro_W1, ro_b1, ro_W2, ro_b2):
    f32 = jnp.float32
    z = nxyz[:, 0].astype(jnp.int32).reshape(N_ATOMS, 1)
    xyz = nxyz[:, 1:4]
    a0 = nbr_list[:, 0].astype(jnp.int32)
    a1 = nbr_list[:, 1].astype(jnp.int32)
    a0p = jnp.pad(a0, (0, EP - N_EDGES))
    a1p = jnp.pad(a1, (0, EP - N_EDGES))
    valid = jnp.pad(jnp.ones((N_EDGES,), f32), (0, EP - N_EDGES)).reshape(EP, 1)
    zeros_blk = jnp.zeros((RPS, F), f32)

    emb_pad = jnp.zeros((F, F), f32).at[:100].set(atom_embed)
    b3 = lambda b: b.reshape(N_CONV, 1, F)
    ef1b, ef2b, nfb = b3(ef1_b), b3(ef2_b), b3(nf_b)
    up1b, up2b = b3(up1_b), b3(up2_b)

    d2w = _sc_d2(xyz[:, 0].ravel(), xyz[:, 1].ravel(), xyz[:, 2].ravel(), a0p, a1p)
    d2 = d2w.reshape(EP, 1)

    we0, we1, we2 = _tc_we(d2, valid, ef1_W, ef1b, ef2_W, ef2b)
    wes = (we0, we1, we2)

    r, nodes = _tc_embed(z, emb_pad, nf_W, nfb)

    for i in range(N_CONV):
        part = _sc_msg(nodes, wes[i], a0p, a1p, zeros_blk)
        want_nodes = i < N_CONV - 1
        res = _tc_update(i, want_nodes, r, part[0, :N_ATOMS], part[1, :N_ATOMS],
                         up1_W, up1b, up2_W, up2b, nf_W, nfb)
        if want_nodes:
            r, nodes = res
        else:
            (r,) = res

    r2 = r.reshape(N_SPECIES * N_CONFS, MOL_SIZE * F)
    S = jnp.tile(jnp.eye(F, dtype=f32), (MOL_SIZE, 1))
    P = jnp.kron(jnp.eye(N_SPECIES, dtype=f32), jnp.ones((1, N_CONFS), f32))
    wcol = weights.reshape(N_SPECIES * N_CONFS, 1)
    mb1 = mol_b1.reshape(1, -1)
    mb2 = mol_b2.reshape(1, -1)
    rb1 = ro_b1.reshape(1, -1)
    rb2 = ro_b2.reshape(1, 1)
    return _tc_readout(r2, S, mol_W1, mb1, mol_W2, mb2, wcol, P,
                       ro_W1, rb1, ro_W2, rb2)


# speedup vs baseline: 3.2777x; 1.1461x over previous
import functools

import jax
import jax.numpy as jnp
from jax import lax
from jax.experimental import pallas as pl
from jax.experimental.pallas import tpu as pltpu
from jax.experimental.pallas import tpu_sc as plsc

N_ATOMS = 10000
N_EDGES = 160000
N_SPECIES = 50
N_CONFS = 10
MOL_SIZE = 20
F = 128
G = 32
N_CONV = 3
CUTOFF = 5.0

NC = 2
NS = 16
NW = NC * NS
EP = 163840
EWM = EP // NW
C = 64
TOT_CH = EP // C
N0 = 102
N1 = (TOT_CH - NS * N0) // NS
NCHMAX = max(N0, N1)
NP = 10112
RPS = NP // NS

_LOG2 = 0.6931471805599453


def _ssp(x):
    return jnp.maximum(x, 0.0) + jnp.log(1.0 + jnp.exp(-jnp.abs(x))) - _LOG2


def _mesh():
    return plsc.VectorSubcoreMesh(
        core_axis_name="c", subcore_axis_name="s", num_cores=NC, num_subcores=NS
    )


def _sc_d2_body(x_hbm, y_hbm, z_hbm, a0_hbm, a1_hbm, out_hbm,
                xv, yv, zv, i0v, i1v, dv):
    cid = lax.axis_index("c")
    sid = lax.axis_index("s")
    wid = cid * NS + sid
    pltpu.sync_copy(x_hbm, xv)
    pltpu.sync_copy(y_hbm, yv)
    pltpu.sync_copy(z_hbm, zv)
    base = wid * EWM
    pltpu.sync_copy(a0_hbm.at[pl.ds(base, EWM)], i0v)
    pltpu.sync_copy(a1_hbm.at[pl.ds(base, EWM)], i1v)

    def body(k, _):
        i0 = i0v[pl.ds(k * 16, 16)]
        i1 = i1v[pl.ds(k * 16, 16)]
        dx = plsc.load_gather(xv, [i0]) - plsc.load_gather(xv, [i1])
        dy = plsc.load_gather(yv, [i0]) - plsc.load_gather(yv, [i1])
        dz = plsc.load_gather(zv, [i0]) - plsc.load_gather(zv, [i1])
        dv[pl.ds(k * 16, 16)] = dx * dx + dy * dy + dz * dz
        return 0

    lax.fori_loop(0, EWM // 16, body, 0)
    pltpu.sync_copy(dv, out_hbm.at[wid])


def _sc_d2(x, y, z, a0p, a1p):
    fn = pl.kernel(
        _sc_d2_body,
        out_type=jax.ShapeDtypeStruct((NW, EWM), jnp.float32),
        mesh=_mesh(),
        scratch_types=[
            pltpu.VMEM((N_ATOMS,), jnp.float32),
            pltpu.VMEM((N_ATOMS,), jnp.float32),
            pltpu.VMEM((N_ATOMS,), jnp.float32),
            pltpu.VMEM((EWM,), jnp.int32),
            pltpu.VMEM((EWM,), jnp.int32),
            pltpu.VMEM((EWM,), jnp.float32),
        ],
        compiler_params=pltpu.CompilerParams(needs_layout_passes=False),
    )
    return fn(x, y, z, a0p, a1p)


def _sc_msg_body(nodes_hbm, we_hbm, idx0_hbm, idx1_hbm, zeros_hbm, out_hbm,
                 ic0_0, ic0_1, ic1_0, ic1_1,
                 rows0_0, rows0_1, rows1_0, rows1_1, web_0, web_1, agg,
                 si_0, si_1, sg0_0, sg0_1, sg1_0, sg1_1, sw_0, sw_1):
    cid = lax.axis_index("c")
    sid = lax.axis_index("s")

    ncc = jnp.where(cid == 0, N0, N1)
    cb = jnp.where(cid == 0, sid * N0, NS * N0 + sid * N1)

    ic0 = (ic0_0, ic0_1)
    ic1 = (ic1_0, ic1_1)
    rows0 = (rows0_0, rows0_1)
    rows1 = (rows1_0, rows1_1)
    web = (web_0, web_1)
    si = (si_0, si_1)
    sg0 = (sg0_0, sg0_1)
    sg1 = (sg1_0, sg1_1)
    sw = (sw_0, sw_1)

    def fire_idx(j, b):
        pltpu.async_copy(idx0_hbm.at[pl.ds((cb + j) * C, C)], ic0[b], si[b])
        pltpu.async_copy(idx1_hbm.at[pl.ds((cb + j) * C, C)], ic1[b], si[b])

    def wait_idx(b):
        pltpu.make_async_copy(idx0_hbm.at[pl.ds(0, C)], ic0[b], si[b]).wait()
        pltpu.make_async_copy(idx1_hbm.at[pl.ds(0, C)], ic1[b], si[b]).wait()

    def fire_gathers(j, b):
        pltpu.async_copy(nodes_hbm.at[ic0[b]], rows0[b], sg0[b])
        pltpu.async_copy(nodes_hbm.at[ic1[b]], rows1[b], sg1[b])
        pltpu.async_copy(we_hbm.at[pl.ds((cb + j) * C, C)], web[b], sw[b])

    def wait_gathers(b):
        pltpu.make_async_copy(nodes_hbm.at[ic0[b]], rows0[b], sg0[b]).wait()
        pltpu.make_async_copy(nodes_hbm.at[ic1[b]], rows1[b], sg1[b]).wait()
        pltpu.make_async_copy(we_hbm.at[pl.ds(0, C)], web[b], sw[b]).wait()

    fire_idx(0, 0)
    fire_idx(1, 1)
    wait_idx(0)
    fire_gathers(0, 0)

    pltpu.sync_copy(zeros_hbm, agg.at[pl.ds(sid * RPS, RPS)])
    plsc.subcore_barrier()

    def super_step(t, _):
        for b in range(2):
            j = 2 * t + b
            nb = 1 - b
            wait_gathers(b)

            @pl.when(j + 1 < ncc)
            def _():
                wait_idx(nb)
                fire_gathers(j + 1, nb)

            def mrow(r, _):
                for v in range(8):
                    w16 = web[b][r, pl.ds(v * 16, 16)]
                    rows0[b][r, pl.ds(v * 16, 16)] = rows0[b][r, pl.ds(v * 16, 16)] * w16
                    rows1[b][r, pl.ds(v * 16, 16)] = rows1[b][r, pl.ds(v * 16, 16)] * w16
                return 0

            lax.fori_loop(0, C, mrow, 0)
            pltpu.sync_copy(rows0[b], agg.at[ic1[b]], add=True)
            pltpu.sync_copy(rows1[b], agg.at[ic0[b]], add=True)

            @pl.when(j + 2 < ncc)
            def _():
                fire_idx(j + 2, b)
        return 0

    lax.fori_loop(0, ncc // 2, super_step, 0)
    plsc.subcore_barrier()
    pltpu.sync_copy(agg.at[pl.ds(sid * RPS, RPS)],
                    out_hbm.at[cid, pl.ds(sid * RPS, RPS)])


def _sc_msg(nodes, we, idx0, idx1, zeros):
    fn = pl.kernel(
        _sc_msg_body,
        out_type=jax.ShapeDtypeStruct((NC, NP, F), jnp.float32),
        mesh=_mesh(),
        scratch_types=[
            pltpu.VMEM((C,), jnp.int32),
            pltpu.VMEM((C,), jnp.int32),
            pltpu.VMEM((C,), jnp.int32),
            pltpu.VMEM((C,), jnp.int32),
            pltpu.VMEM((C, F), jnp.float32),
            pltpu.VMEM((C, F), jnp.float32),
            pltpu.VMEM((C, F), jnp.float32),
            pltpu.VMEM((C, F), jnp.float32),
            pltpu.VMEM((C, F), jnp.float32),
            pltpu.VMEM((C, F), jnp.float32),
            pltpu.VMEM_SHARED((NP, F), jnp.float32),
            pltpu.SemaphoreType.DMA,
            pltpu.SemaphoreType.DMA,
            pltpu.SemaphoreType.DMA,
            pltpu.SemaphoreType.DMA,
            pltpu.SemaphoreType.DMA,
            pltpu.SemaphoreType.DMA,
            pltpu.SemaphoreType.DMA,
            pltpu.SemaphoreType.DMA,
        ],
        compiler_params=pltpu.CompilerParams(needs_layout_passes=False),
    )
    return fn(nodes, we, idx0, idx1, zeros)


_EB = 1024
_AB = 1000


def _tc_we_body(d2_ref, valid_ref, ef1w_ref, ef1b_ref, ef2w_ref, ef2b_ref,
                o0, o1, o2):
    d = jnp.sqrt(d2_ref[...])
    width = CUTOFF / (G - 1)
    coeff = -0.5 / (width * width)
    offs = lax.broadcasted_iota(jnp.int32, (_EB, G), 1).astype(jnp.float32) * width
    g = jnp.exp(coeff * (d - offs) ** 2)
    valid = valid_ref[...]
    outs = (o0, o1, o2)
    for i in range(N_CONV):
        h = _ssp(jnp.dot(g, ef1w_ref[i], preferred_element_type=jnp.float32)
                 + ef1b_ref[i])
        outs[i][...] = (jnp.dot(h, ef2w_ref[i], preferred_element_type=jnp.float32)
                        + ef2b_ref[i]) * valid


def _tc_we(d2, valid, ef1_W, ef1_b, ef2_W, ef2_b):
    nblk = EP // _EB
    full = lambda s: pl.BlockSpec(s, lambda i: (0,) * len(s))
    out = jax.ShapeDtypeStruct((EP, F), jnp.float32)
    return pl.pallas_call(
        _tc_we_body,
        grid=(nblk,),
        in_specs=[
            pl.BlockSpec((_EB, 1), lambda i: (i, 0)),
            pl.BlockSpec((_EB, 1), lambda i: (i, 0)),
            full((N_CONV, G, F)), full((N_CONV, 1, F)),
            full((N_CONV, F, F)), full((N_CONV, 1, F)),
        ],
        out_specs=[pl.BlockSpec((_EB, F), lambda i: (i, 0))] * 3,
        out_shape=[out, out, out],
    )(d2, valid, ef1_W, ef1_b, ef2_W, ef2_b)


def _tc_embed_body(z_ref, emb_ref, nfw_ref, nfb_ref, r_out, nodes_out):
    zb = z_ref[...]
    onehot = (zb == lax.broadcasted_iota(jnp.int32, (_AB, F), 1)).astype(jnp.float32)
    r = jnp.dot(onehot, emb_ref[...], preferred_element_type=jnp.float32)
    r_out[...] = r
    nodes_out[...] = (jnp.dot(r, nfw_ref[0], preferred_element_type=jnp.float32)
                      + nfb_ref[0])


def _tc_embed(z, emb_pad, nf_W, nf_b):
    nblk = N_ATOMS // _AB
    full = lambda s: pl.BlockSpec(s, lambda i: (0,) * len(s))
    out = jax.ShapeDtypeStruct((N_ATOMS, F), jnp.float32)
    return pl.pallas_call(
        _tc_embed_body,
        grid=(nblk,),
        in_specs=[
            pl.BlockSpec((_AB, 1), lambda i: (i, 0)),
            full((F, F)), full((N_CONV, F, F)), full((N_CONV, 1, F)),
        ],
        out_specs=[pl.BlockSpec((_AB, F), lambda i: (i, 0))] * 2,
        out_shape=[out, out],
    )(z, emb_pad, nf_W, nf_b)


def _tc_update_body(i, want_nodes, r_ref, q0_ref, q1_ref,
                    up1w_ref, up1b_ref, up2w_ref, up2b_ref,
                    nfw_ref, nfb_ref, *outs):
    agg = q0_ref[...] + q1_ref[...]
    h = _ssp(jnp.dot(agg, up1w_ref[i], preferred_element_type=jnp.float32)
             + up1b_ref[i])
    dr = jnp.dot(h, up2w_ref[i], preferred_element_type=jnp.float32) + up2b_ref[i]
    rn = r_ref[...] + dr
    outs[0][...] = rn
    if want_nodes:
        outs[1][...] = (jnp.dot(rn, nfw_ref[i + 1], preferred_element_type=jnp.float32)
                        + nfb_ref[i + 1])


def _tc_update(i, want_nodes, r, q0, q1, up1_W, up1_b, up2_W, up2_b, nf_W, nf_b):
    nblk = N_ATOMS // _AB
    full = lambda s: pl.BlockSpec(s, lambda i: (0,) * len(s))
    ab = pl.BlockSpec((_AB, F), lambda i: (i, 0))
    out = jax.ShapeDtypeStruct((N_ATOMS, F), jnp.float32)
    return pl.pallas_call(
        functools.partial(_tc_update_body, i, want_nodes),
        grid=(nblk,),
        in_specs=[ab, ab, ab,
                  full((N_CONV, F, F)), full((N_CONV, 1, F)),
                  full((N_CONV, F, F)), full((N_CONV, 1, F)),
                  full((N_CONV, F, F)), full((N_CONV, 1, F))],
        out_specs=[ab] * (2 if want_nodes else 1),
        out_shape=[out] * (2 if want_nodes else 1),
    )(r, q0, q1, up1_W, up1_b, up2_W, up2_b, nf_W, nf_b)


def _tc_readout_body(r2_ref, S_ref, mw1_ref, mb1_ref, mw2_ref, mb2_ref,
                     wcol_ref, P_ref, rw1_ref, rb1_ref, rw2_ref, rb2_ref, o_ref):
    pooled = jnp.dot(r2_ref[...], S_ref[...], preferred_element_type=jnp.float32)
    h = _ssp(jnp.dot(pooled, mw1_ref[...], preferred_element_type=jnp.float32)
             + mb1_ref[...])
    mol = jnp.dot(h, mw2_ref[...], preferred_element_type=jnp.float32) + mb2_ref[...]
    molw = mol * wcol_ref[...]
    fp = jnp.dot(P_ref[...], molw, preferred_element_type=jnp.float32)
    hh = _ssp(jnp.dot(fp, rw1_ref[...], preferred_element_type=jnp.float32)
              + rb1_ref[...])
    o_ref[...] = jnp.dot(hh, rw2_ref[...], preferred_element_type=jnp.float32) + rb2_ref[...]


def _tc_readout(r2, S, mw1, mb1, mw2, mb2, wcol, P, rw1, rb1, rw2, rb2):
    return pl.pallas_call(
        _tc_readout_body,
        out_shape=jax.ShapeDtypeStruct((N_SPECIES, 1), jnp.float32),
    )(r2, S, mw1, mb1, mw2, mb2, wcol, P, rw1, rb1, rw2, rb2)


def kernel(nxyz, nbr_list, weights, atom_embed,
           ef1_W, ef1_b, ef2_W, ef2_b, nf_W, nf_b,
           up1_W, up1_b, up2_W, up2_b,
           mol_W1, mol_b1, mol_W2, mol_b2,
           ro_W1, ro_b1, ro_W2, ro_b2):
    f32 = jnp.float32
    z = nxyz[:, 0].astype(jnp.int32).reshape(N_ATOMS, 1)
    xyz = nxyz[:, 1:4]
    a0 = nbr_list[:, 0].astype(jnp.int32)
    a1 = nbr_list[:, 1].astype(jnp.int32)
    a0p = jnp.pad(a0, (0, EP - N_EDGES))
    a1p = jnp.pad(a1, (0, EP - N_EDGES))
    valid = jnp.pad(jnp.ones((N_EDGES,), f32), (0, EP - N_EDGES)).reshape(EP, 1)
    zeros_blk = jnp.zeros((RPS, F), f32)

    emb_pad = jnp.zeros((F, F), f32).at[:100].set(atom_embed)
    b3 = lambda b: b.reshape(N_CONV, 1, F)
    ef1b, ef2b, nfb = b3(ef1_b), b3(ef2_b), b3(nf_b)
    up1b, up2b = b3(up1_b), b3(up2_b)

    d2w = _sc_d2(xyz[:, 0].ravel(), xyz[:, 1].ravel(), xyz[:, 2].ravel(), a0p, a1p)
    d2 = d2w.reshape(EP, 1)

    we0, we1, we2 = _tc_we(d2, valid, ef1_W, ef1b, ef2_W, ef2b)
    wes = (we0, we1, we2)

    r, nodes = _tc_embed(z, emb_pad, nf_W, nfb)

    for i in range(N_CONV):
        part = _sc_msg(nodes, wes[i], a0p, a1p, zeros_blk)
        want_nodes = i < N_CONV - 1
        res = _tc_update(i, want_nodes, r, part[0, :N_ATOMS], part[1, :N_ATOMS],
                         up1_W, up1b, up2_W, up2b, nf_W, nfb)
        if want_nodes:
            r, nodes = res
        else:
            (r,) = res

    r2 = r.reshape(N_SPECIES * N_CONFS, MOL_SIZE * F)
    S = jnp.tile(jnp.eye(F, dtype=f32), (MOL_SIZE, 1))
    P = jnp.kron(jnp.eye(N_SPECIES, dtype=f32), jnp.ones((1, N_CONFS), f32))
    wcol = weights.reshape(N_SPECIES * N_CONFS, 1)
    mb1 = mol_b1.reshape(1, -1)
    mb2 = mol_b2.reshape(1, -1)
    rb1 = ro_b1.reshape(1, -1)
    rb2 = ro_b2.reshape(1, 1)
    return _tc_readout(r2, S, mol_W1, mb1, mol_W2, mb2, wcol, P,
                       ro_W1, rb1, ro_W2, rb2)
